# baseline (speedup 1.0000x reference)
import functools

import jax
import jax.numpy as jnp
from jax import lax
from jax.experimental import pallas as pl
from jax.experimental.pallas import tpu as pltpu
from jax.experimental.pallas import tpu_sc as plsc

NC = 2
NS = 16
NW = NC * NS

D = 64
DP = 128
SEG = 128
NBUF = 2


def _sc_embed(batch, seq, n_scores, words_pad, scores_pad, widx, sidx):
    per_w = batch // NW
    rem = seq - SEG
    dh_n = D // 8
    bh_n = batch // SEG

    mesh = plsc.VectorSubcoreMesh(
        core_axis_name="c", subcore_axis_name="s",
        num_cores=NC, num_subcores=NS,
    )

    @functools.partial(
        pl.kernel,
        out_type=jax.ShapeDtypeStruct((seq, dh_n, bh_n, 8, SEG), jnp.float32),
        mesh=mesh,
        scratch_types=[
            pltpu.VMEM_SHARED((n_scores, DP), jnp.float32),
            pltpu.VMEM((SEG, SEG), jnp.int32),
            pltpu.VMEM((SEG, rem), jnp.int32),
            pltpu.VMEM((SEG, SEG), jnp.int32),
            pltpu.VMEM((SEG, rem), jnp.int32),
            pltpu.VMEM((NBUF, SEG), jnp.int32),
            pltpu.VMEM((NBUF, SEG), jnp.int32),
            pltpu.VMEM((NBUF, SEG, DP), jnp.float32),
            pltpu.VMEM((NBUF, dh_n, 8, SEG), jnp.float32),
            [pltpu.SemaphoreType.DMA] * NBUF,
            [pltpu.SemaphoreType.DMA] * NBUF,
            [pltpu.SemaphoreType.DMA] * NBUF,
        ],
        compiler_params=pltpu.CompilerParams(
            needs_layout_passes=False, use_tc_tiling_on_sc=True),
    )
    def k(words_hbm, stab_hbm, widx_hbm, sidx_hbm, out_hbm,
          stab_sh, wa, wb, sa, sb, widx_r, sidx_r, rows_v, trans_v,
          semW, semS, semO):
        cid = lax.axis_index("c")
        sid = lax.axis_index("s")
        wid = sid * NC + cid
        base_b = wid * per_w

        @pl.when(sid == 0)
        def _():
            pltpu.sync_copy(stab_hbm, stab_sh)

        plsc.subcore_barrier()

        pltpu.sync_copy(widx_hbm.at[pl.ds(base_b, SEG), pl.ds(0, SEG)], wa)
        pltpu.sync_copy(widx_hbm.at[pl.ds(base_b, SEG), pl.ds(SEG, rem)], wb)
        pltpu.sync_copy(sidx_hbm.at[pl.ds(base_b, SEG), pl.ds(0, SEG)], sa)
        pltpu.sync_copy(sidx_hbm.at[pl.ds(base_b, SEG), pl.ds(SEG, rem)], sb)

        iota = lax.iota(jnp.int32, 16)
        dW = {p: [] for p in range(NBUF)}
        dS = {p: [] for p in range(NBUF)}
        dO = {p: [] for p in range(NBUF)}

        def repack(col_a, col_b, col_sel, ra, rb, dst, p):
            for g in range(8):
                bi = iota + 16 * g
                va = plsc.load_gather(ra, [bi, col_a])
                vb = plsc.load_gather(rb, [bi, col_b])
                dst[p, pl.ds(16 * g, 16)] = jnp.where(col_sel, va, vb)

        def front(l, p):
            col_sel = jnp.full((16,), l, jnp.int32) < SEG
            col_a = jnp.full((16,), lax.rem(l, SEG), jnp.int32)
            col_b = jnp.full((16,), lax.max(l - SEG, 0), jnp.int32)
            repack(col_a, col_b, col_sel, wa, wb, widx_r, p)
            repack(col_a, col_b, col_sel, sa, sb, sidx_r, p)
            dW[p].append(pltpu.async_copy(
                words_hbm.at[widx_r.at[p]], rows_v.at[p], semW[p]))

        def drain(descs):
            for d in descs:
                d.wait()
            descs.clear()

        def drain_recon(dst_slice, hbm_src, sem):
            pltpu.make_async_copy(hbm_src, dst_slice, sem).wait()

        def finish(l, q, drain_o, real_w):
            if drain_o:
                drain_recon(trans_v.at[q],
                            out_hbm.at[0, pl.ds(0, dh_n), 0], semO[q])
            dO[q].clear()
            if real_w:
                drain(dW[q])
            else:
                dW[q].clear()
                drain_recon(rows_v.at[q],
                            words_hbm.at[pl.ds(0, SEG)], semW[q])
            dS[q].append(pltpu.async_copy(
                stab_sh.at[sidx_r.at[q]], rows_v.at[q], semS[q], add=True))
            drain(dS[q])

            page = rows_v.at[q]

            def dbody(d, carry):
                dh = lax.div(d, 8)
                dl = lax.rem(d, 8)
                dcol = jnp.full((16,), d, jnp.int32)
                for g in range(8):
                    v = plsc.load_gather(page, [iota + 16 * g, dcol])
                    trans_v[q, dh, dl, pl.ds(16 * g, 16)] = v
                return carry

            lax.fori_loop(0, D, dbody, 0, unroll=False)
            dO[q].append(pltpu.async_copy(
                trans_v.at[q], out_hbm.at[l, pl.ds(0, dh_n), wid], semO[q]))

        front(0, 0)
        front(1, 1)
        finish(0, 0, drain_o=False, real_w=True)
        front(2, 0)
        finish(1, 1, drain_o=False, real_w=True)

        def body(i, carry):
            l1 = 3 + 2 * i
            front(l1, 1)
            finish(l1 - 1, 0, drain_o=True, real_w=False)
            front(l1 + 1, 0)
            finish(l1, 1, drain_o=True, real_w=True)
            return carry

        lax.fori_loop(0, (seq - 4) // 2, body, 0, unroll=False)

        front(seq - 1, 1)
        finish(seq - 2, 0, drain_o=True, real_w=False)
        finish(seq - 1, 1, drain_o=True, real_w=True)
        for q in range(NBUF):
            drain(dO[q])

    return k(words_pad, scores_pad, widx, sidx)


def kernel(input_ids, scores_ids, words_emb, scores_emb):
    batch, seq = input_ids.shape
    words_pad = jnp.pad(words_emb, ((0, 0), (0, DP - D)))
    scores_pad = jnp.pad(scores_emb, ((0, 0), (0, DP - D)))
    p5 = _sc_embed(batch, seq, scores_emb.shape[0],
                   words_pad, scores_pad,
                   input_ids.astype(jnp.int32),
                   scores_ids.astype(jnp.int32))
    return p5.transpose(2, 4, 0, 1, 3).reshape(batch, seq, D)


# speedup vs baseline: 1.9451x; 1.9451x over previous
import functools

import jax
import jax.numpy as jnp
from jax import lax
from jax.experimental import pallas as pl
from jax.experimental.pallas import tpu as pltpu
from jax.experimental.pallas import tpu_sc as plsc

NC = 2
NS = 16
NW = NC * NS

D = 64
DP = 128
M = 2
SEG = 128
NBUF = 2


def _sc_embed(batch, seq, n_scores, words_pad, scores_pad, widx, sidx):
    per_w = batch // NW
    n_chunks = per_w // M
    rem = seq - SEG

    mesh = plsc.VectorSubcoreMesh(
        core_axis_name="c", subcore_axis_name="s",
        num_cores=NC, num_subcores=NS,
    )

    @functools.partial(
        pl.kernel,
        out_type=jax.ShapeDtypeStruct((batch, seq, DP), jnp.float32),
        mesh=mesh,
        scratch_types=[
            pltpu.VMEM_SHARED((n_scores, DP), jnp.float32),
            pltpu.VMEM((NBUF, M, seq), jnp.int32),
            pltpu.VMEM((NBUF, M, seq), jnp.int32),
            pltpu.VMEM((NBUF, M, seq, DP), jnp.float32),
            [pltpu.SemaphoreType.DMA] * NBUF,
            [pltpu.SemaphoreType.DMA] * NBUF,
            [pltpu.SemaphoreType.DMA] * NBUF,
            [pltpu.SemaphoreType.DMA] * NBUF,
        ],
        compiler_params=pltpu.CompilerParams(
            needs_layout_passes=False, use_tc_tiling_on_sc=True),
    )
    def k(words_hbm, stab_hbm, widx_hbm, sidx_hbm, out_hbm,
          stab_sh, widx_v, sidx_v, rows_v, semI, semW, semS, semO):
        cid = lax.axis_index("c")
        sid = lax.axis_index("s")
        wid = sid * NC + cid
        base_b = wid * per_w

        @pl.when(sid == 0)
        def _():
            pltpu.sync_copy(stab_hbm, stab_sh)

        plsc.subcore_barrier()

        dI = {b: [] for b in range(NBUF)}
        dW = {b: [] for b in range(NBUF)}
        dS = {b: [] for b in range(NBUF)}
        dO = {b: [] for b in range(NBUF)}

        def drain(descs):
            for d in descs:
                d.wait()
            descs.clear()

        def fire_idx(c):
            b = c % NBUF
            b0 = base_b + c * M
            dI[b].append(pltpu.async_copy(
                widx_hbm.at[pl.ds(b0, M)], widx_v.at[b], semI[b]))
            dI[b].append(pltpu.async_copy(
                sidx_hbm.at[pl.ds(b0, M)], sidx_v.at[b], semI[b]))

        def fire_gathers(c, idx_ref, src, sem_list, descs, add):
            b = c % NBUF
            for i in range(M):
                for off, ln in ((0, SEG), (SEG, rem)):
                    descs[b].append(pltpu.async_copy(
                        src.at[idx_ref.at[b, i, pl.ds(off, ln)]],
                        rows_v.at[b, i, pl.ds(off, ln)],
                        sem_list[b], add=add))

        def fire_out(c):
            b = c % NBUF
            b0 = base_b + c * M
            dO[b].append(pltpu.async_copy(
                rows_v.at[b], out_hbm.at[pl.ds(b0, M)], semO[b]))

        fire_idx(0)
        for c in range(n_chunks):
            b = c % NBUF
            drain(dO[b])
            drain(dI[b])
            fire_gathers(c, widx_v, words_hbm, semW, dW, False)
            if c == 0:
                fire_idx(1)
            else:
                p = c - 1
                d = p % NBUF
                drain(dW[d])
                fire_gathers(p, sidx_v, stab_sh, semS, dS, True)
                drain(dS[d])
                if c + 1 < n_chunks:
                    fire_idx(c + 1)
                fire_out(p)
        p = n_chunks - 1
        d = p % NBUF
        drain(dW[d])
        fire_gathers(p, sidx_v, stab_sh, semS, dS, True)
        drain(dS[d])
        fire_out(p)
        for b in range(NBUF):
            drain(dO[b])

    return k(words_pad, scores_pad, widx, sidx)


def kernel(input_ids, scores_ids, words_emb, scores_emb):
    batch, seq = input_ids.shape
    words_pad = jnp.pad(words_emb, ((0, 0), (0, DP - D)))
    scores_pad = jnp.pad(scores_emb, ((0, 0), (0, DP - D)))
    out_pad = _sc_embed(batch, seq, scores_emb.shape[0],
                        words_pad, scores_pad,
                        input_ids.astype(jnp.int32),
                        scores_ids.astype(jnp.int32))
    return out_pad[:, :, :D]
